# baseline (speedup 1.0000x reference)
import functools

import jax
import jax.numpy as jnp
from jax import lax
from jax.experimental import pallas as pl
from jax.experimental.pallas import tpu as pltpu
from jax.experimental.pallas import tpu_sc as plsc

N_V = 2
N_D = 128
BATCH = 16384
HIST = 200
B_TOK = BATCH * HIST
NC, NS, L = 2, 16, 16
NW = NC * NS
B_PER_W = B_TOK // NW
CHUNK = 256
NBUF = 2
IBLK = 51200
N_OUTER = B_PER_W // IBLK
N_CHUNK = IBLK // CHUNK
N_ROUND = N_CHUNK // NBUF
GRP = CHUNK // L


def _lane_bcast(vec, u):
    idx = jnp.full((L, 1), u, jnp.int32)
    dnums = lax.GatherDimensionNumbers(
        offset_dims=(), collapsed_slice_dims=(0,), start_index_map=(0,))
    return lax.gather(vec, idx, dnums, (1,),
                      mode=lax.GatherScatterMode.PROMISE_IN_BOUNDS)


@functools.partial(
    pl.kernel,
    mesh=plsc.VectorSubcoreMesh(core_axis_name="c", subcore_axis_name="s"),
    out_type=jax.ShapeDtypeStruct((B_TOK, N_D), jnp.float32),
    scratch_types=[
        pltpu.VMEM((2, N_D), jnp.float32),
        pltpu.VMEM((IBLK,), jnp.int32),
    ]
    + [pltpu.VMEM((CHUNK, N_D), jnp.float32) for _ in range(NBUF)]
    + [pltpu.SemaphoreType.DMA for _ in range(NBUF)],
)
def _sc_lookup(table_hbm, idx_hbm, out_hbm, table_v, idx_v, *bufs):
    rows = bufs[:NBUF]
    sems = bufs[NBUF:]
    wid = lax.axis_index("s") * NC + lax.axis_index("c")
    base = wid * B_PER_W

    pltpu.sync_copy(table_hbm, table_v)
    w0 = [table_v[0, pl.ds(16 * j, 16)] for j in range(8)]
    dif = [table_v[1, pl.ds(16 * j, 16)] - w0[j] for j in range(8)]

    def compute_fire(g, b, obase):
        cbase = g * CHUNK

        @plsc.parallel_loop(0, GRP, unroll=4)
        def grp_body(t0):
            iv16 = idx_v[pl.ds(cbase + t0 * L, L)]
            for u in range(L):
                f = _lane_bcast(iv16, u).astype(jnp.float32)
                t = t0 * L + u
                for j in range(8):
                    rows[b][t, pl.ds(16 * j, 16)] = w0[j] + f * dif[j]
        pltpu.async_copy(rows[b], out_hbm.at[pl.ds(obase + cbase, CHUNK)],
                         sems[b])

    def drain(b, obase):
        pltpu.make_async_copy(rows[b], out_hbm.at[pl.ds(obase, CHUNK)],
                              sems[b]).wait()

    def outer_body(o, carry):
        obase = base + o * IBLK
        pltpu.sync_copy(idx_hbm.at[pl.ds(obase, IBLK)], idx_v)
        for b in range(NBUF):
            compute_fire(b, b, obase)

        def round_body(r, c2):
            for b in range(NBUF):
                drain(b, obase)
                compute_fire(r * NBUF + b, b, obase)
            return c2

        lax.fori_loop(1, N_ROUND, round_body, 0)
        for b in range(NBUF):
            drain(b, obase)
        return carry

    lax.fori_loop(0, N_OUTER, outer_body, 0)


def kernel(input, weight_mean, weight_var):
    del weight_var
    idx_flat = input.reshape(B_TOK)
    out_flat = _sc_lookup(weight_mean, idx_flat)
    return out_flat.reshape(BATCH, HIST, N_D)


# speedup vs baseline: 1.0127x; 1.0127x over previous
import functools

import jax
import jax.numpy as jnp
from jax import lax
from jax.experimental import pallas as pl
from jax.experimental.pallas import tpu as pltpu
from jax.experimental.pallas import tpu_sc as plsc

N_V = 2
N_D = 128
BATCH = 16384
HIST = 200
B_TOK = BATCH * HIST
NC, NS, L = 2, 16, 16
NW = NC * NS
B_PER_W = B_TOK // NW
CHUNK = 256
NBUF = 2
IBLK = 51200
N_OUTER = B_PER_W // IBLK
N_CHUNK = IBLK // CHUNK
N_ROUND = N_CHUNK // NBUF
GRP = CHUNK // L


def _lane_bcast(vec, u):
    idx = jnp.full((L, 1), u, jnp.int32)
    dnums = lax.GatherDimensionNumbers(
        offset_dims=(), collapsed_slice_dims=(0,), start_index_map=(0,))
    return lax.gather(vec, idx, dnums, (1,),
                      mode=lax.GatherScatterMode.PROMISE_IN_BOUNDS)


@functools.partial(
    pl.kernel,
    mesh=plsc.VectorSubcoreMesh(core_axis_name="c", subcore_axis_name="s"),
    out_type=jax.ShapeDtypeStruct((B_TOK, N_D), jnp.float32),
    scratch_types=[
        pltpu.VMEM((2, N_D), jnp.float32),
        pltpu.VMEM((IBLK,), jnp.int32),
    ]
    + [pltpu.VMEM((CHUNK, N_D), jnp.float32) for _ in range(NBUF)]
    + [pltpu.SemaphoreType.DMA for _ in range(NBUF)],
)
def _sc_lookup(table_hbm, idx_hbm, out_hbm, table_v, idx_v, *bufs):
    rows = bufs[:NBUF]
    sems = bufs[NBUF:]
    wid = lax.axis_index("s") * NC + lax.axis_index("c")
    base = wid * B_PER_W

    pltpu.sync_copy(table_hbm, table_v)
    w0 = [table_v[0, pl.ds(16 * j, 16)] for j in range(8)]
    dif = [table_v[1, pl.ds(16 * j, 16)] - w0[j] for j in range(8)]

    def compute_fire(g, b, obase):
        cbase = g * CHUNK

        @plsc.parallel_loop(0, GRP, unroll=2)
        def grp_body(t0):
            iv16 = idx_v[pl.ds(cbase + t0 * L, L)]
            for u in range(L):
                f = _lane_bcast(iv16, u).astype(jnp.float32)
                t = t0 * L + u
                for j in range(8):
                    rows[b][t, pl.ds(16 * j, 16)] = w0[j] + f * dif[j]
        pltpu.async_copy(rows[b], out_hbm.at[pl.ds(obase + cbase, CHUNK)],
                         sems[b])

    def drain(b, obase):
        pltpu.make_async_copy(rows[b], out_hbm.at[pl.ds(obase, CHUNK)],
                              sems[b]).wait()

    def outer_body(o, carry):
        obase = base + o * IBLK
        pltpu.sync_copy(idx_hbm.at[pl.ds(obase, IBLK)], idx_v)
        for b in range(NBUF):
            compute_fire(b, b, obase)

        def round_body(r, c2):
            for b in range(NBUF):
                drain(b, obase)
                compute_fire(r * NBUF + b, b, obase)
            return c2

        lax.fori_loop(1, N_ROUND, round_body, 0)
        for b in range(NBUF):
            drain(b, obase)
        return carry

    lax.fori_loop(0, N_OUTER, outer_body, 0)


def kernel(input, weight_mean, weight_var):
    del weight_var
    idx_flat = input.reshape(B_TOK)
    out_flat = _sc_lookup(weight_mean, idx_flat)
    return out_flat.reshape(BATCH, HIST, N_D)
